# Initial kernel scaffold; baseline (speedup 1.0000x reference)
import jax
import jax.numpy as jnp
from jax.experimental import pallas as pl


def kernel(t_cls_scores, t_centernesses):
    raise NotImplementedError("write your pallas kernel here")


# speedup vs baseline: 1.2094x; 1.2094x over previous
import functools

import jax
import jax.numpy as jnp
from jax import lax
from jax.experimental import pallas as pl
from jax.experimental.pallas import tpu as pltpu
from jax.experimental.pallas import tpu_sc as plsc

_L = 16
_NSUB = 16
_BLK = 512


def _dense_stage(cls_scores, cent):
  n, nc = cls_scores.shape
  grid = n // _BLK

  def body(cls_ref, cent_ref, ts_ref, joint_ref):
    probs = jax.nn.sigmoid(cls_ref[...])
    s = jnp.max(probs, axis=1, keepdims=True)
    ts_ref[...] = s
    joint_ref[...] = jax.nn.sigmoid(cent_ref[...]) * s

  ts2, joint2 = pl.pallas_call(
      body,
      grid=(grid,),
      in_specs=[pl.BlockSpec((_BLK, nc), lambda i: (i, 0)),
                pl.BlockSpec((_BLK, 1), lambda i: (i, 0))],
      out_specs=[pl.BlockSpec((_BLK, 1), lambda i: (i, 0)),
                 pl.BlockSpec((_BLK, 1), lambda i: (i, 0))],
      out_shape=[jax.ShapeDtypeStruct((n, 1), jnp.float32),
                 jax.ShapeDtypeStruct((n, 1), jnp.float32)],
  )(cls_scores, cent)
  return ts2.reshape(-1), joint2.reshape(-1)


def _select_stage(ts, k):
  n = ts.shape[0]
  rows = n // _NSUB
  nv = rows // _L

  mesh = plsc.VectorSubcoreMesh(
      core_axis_name="c", subcore_axis_name="s", num_cores=1)

  out_type = (
      jax.ShapeDtypeStruct((n,), jnp.int32),
      jax.ShapeDtypeStruct((n,), jnp.int32),
      jax.ShapeDtypeStruct((_L,), jnp.float32),
  )
  scratch = [
      pltpu.VMEM((rows,), jnp.float32),
      pltpu.VMEM((rows,), jnp.int32),
      pltpu.VMEM((rows,), jnp.int32),
      pltpu.VMEM((_L * 256,), jnp.int32),
      pltpu.VMEM((_L * 256,), jnp.int32),
      pltpu.VMEM((512,), jnp.int32),
      pltpu.VMEM((_NSUB * 512,), jnp.int32),
      pltpu.VMEM((512,), jnp.int32),
      pltpu.VMEM((_L,), jnp.float32),
      pltpu.VMEM((_NSUB * _L,), jnp.float32),
      pltpu.VMEM((_L,), jnp.float32),
      pltpu.VMEM_SHARED((4, _NSUB * 512), jnp.int32),
      pltpu.VMEM_SHARED((_NSUB * _L,), jnp.float32),
  ]

  @functools.partial(
      pl.kernel, out_type=out_type, mesh=mesh, scratch_types=scratch,
      compiler_params=pltpu.CompilerParams(needs_layout_passes=False))
  def sel(ts_hbm, pos_hbm, neg_hbm, scal_hbm,
          ts_v, pos_v, neg_v, hp_v, hn_v, red_v, allh_v, gh_v,
          st_v, alls_v, scal_v, shist, sstat):
    sid = lax.axis_index("s")
    base = sid * rows
    lane = lax.iota(jnp.int32, _L)
    ones = jnp.ones((_L,), jnp.int32)
    zi = jnp.zeros((_L,), jnp.int32)
    zf = jnp.zeros((_L,), jnp.float32)

    pltpu.sync_copy(ts_hbm.at[pl.ds(base, rows)], ts_v)

    kp = jnp.int32(k)
    kn = jnp.int32(k)
    pp = jnp.int32(0)
    pn = jnp.int32(0)
    for r in range(4):
      sh = 24 - 8 * r

      def zb(i, _):
        hp_v[pl.ds(i * _L, _L)] = zi
        hn_v[pl.ds(i * _L, _L)] = zi
        return 0
      lax.fori_loop(0, 256, zb, 0)

      if r == 0:
        def sc0(i, _):
          bits = lax.bitcast_convert_type(ts_v[pl.ds(i * _L, _L)], jnp.int32)
          byte = (bits >> sh) & 255
          plsc.addupdate_scatter(hp_v, [lane * 256 + byte], ones)
          return 0
        lax.fori_loop(0, nv, sc0, 0)
      else:
        mh = jnp.int32(-(1 << (sh + 8)))
        pph = pp
        pnh = pn

        def scr(i, _):
          bits = lax.bitcast_convert_type(ts_v[pl.ds(i * _L, _L)], jnp.int32)
          byte = (bits >> sh) & 255
          idx = lane * 256 + byte
          hi = bits & mh
          plsc.addupdate_scatter(hp_v, [idx], ones, mask=(hi == pph))
          plsc.addupdate_scatter(hn_v, [idx], ones, mask=(hi == pnh))
          return 0
        lax.fori_loop(0, nv, scr, 0)

      def lr(j, _):
        accp = zi
        accn = zi
        for l in range(_L):
          accp = accp + hp_v[pl.ds(l * 256 + j * _L, _L)]
          accn = accn + hn_v[pl.ds(l * 256 + j * _L, _L)]
        red_v[pl.ds(j * _L, _L)] = accp
        red_v[pl.ds(256 + j * _L, _L)] = accn
        return 0
      lax.fori_loop(0, 16, lr, 0)

      pltpu.sync_copy(red_v, shist.at[r, pl.ds(sid * 512, 512)])
      plsc.subcore_barrier()
      pltpu.sync_copy(shist.at[r], allh_v)

      def gm(j, _):
        accp = zi
        accn = zi
        for ss in range(_NSUB):
          accp = accp + allh_v[pl.ds(ss * 512 + j * _L, _L)]
          accn = accn + allh_v[pl.ds(ss * 512 + 256 + j * _L, _L)]
        gh_v[pl.ds(j * _L, _L)] = accp
        gh_v[pl.ds(256 + j * _L, _L)] = accn
        return 0
      lax.fori_loop(0, 16, gm, 0)

      noff = 0 if r == 0 else 256

      def htot(off):
        def tb(j, acc):
          return acc + gh_v[pl.ds(off + j * _L, _L)]
        return jnp.sum(lax.fori_loop(0, 16, tb, zi))

      total_p = htot(0)
      total_n = total_p if r == 0 else htot(256)

      def mb_desc(j, carry):
        cnt, hsum, rowpref = carry
        h = gh_v[pl.ds(j * _L, _L)]
        cums = jnp.cumsum(h)
        pref_lt = rowpref + cums - h
        m = (total_p - pref_lt) >= kp
        return (cnt + plsc.all_reduce_population_count(m),
                hsum + jnp.sum(jnp.where(m, h, 0)),
                rowpref + jnp.sum(h))
      cntp, hsump, _ = lax.fori_loop(
          0, 16, mb_desc, (zi, jnp.int32(0), jnp.int32(0)))
      bp = cntp - 1
      abovep = total_p - hsump

      def mb_asc(j, carry):
        cnt, hsum, rowpref = carry
        h = gh_v[pl.ds(noff + j * _L, _L)]
        cums = jnp.cumsum(h)
        m = (rowpref + cums) >= kn
        return (cnt + plsc.all_reduce_population_count(m),
                hsum + jnp.sum(jnp.where(m, h, 0)),
                rowpref + jnp.sum(h))
      cntn, hsumn, _ = lax.fori_loop(
          0, 16, mb_asc, (zi, jnp.int32(0), jnp.int32(0)))
      bn = 256 - cntn
      belown = total_n - hsumn

      kp = kp - abovep
      pp = pp | (bp << sh)
      kn = kn - belown
      pn = pn | (bn << sh)

    def cnt_body(i, carry):
      cp, cn, sg, st = carry
      v = ts_v[pl.ds(i * _L, _L)]
      bits = lax.bitcast_convert_type(v, jnp.int32)
      cp = cp + (bits == pp).astype(jnp.int32)
      cn = cn + (bits == pn).astype(jnp.int32)
      sg = sg + jnp.where(bits > pp, v, 0.0)
      st = st + v
      return (cp, cn, sg, st)
    cpv, cnv, sgv, stv = lax.fori_loop(0, nv, cnt_body, (zi, zi, zf, zf))
    cposf = jnp.sum(cpv).astype(jnp.float32)
    cnegf = jnp.sum(cnv).astype(jnp.float32)
    sgt = jnp.sum(sgv)
    stot = jnp.sum(stv)

    stats = (cposf * (lane == 0).astype(jnp.float32)
             + cnegf * (lane == 1).astype(jnp.float32)
             + sgt * (lane == 2).astype(jnp.float32)
             + stot * (lane == 3).astype(jnp.float32))
    st_v[...] = stats
    pltpu.sync_copy(st_v, sstat.at[pl.ds(sid * _L, _L)])
    plsc.subcore_barrier()
    pltpu.sync_copy(sstat, alls_v)

    colp = plsc.load_gather(alls_v, [lane * _L])
    coln = plsc.load_gather(alls_v, [lane * _L + 1])
    colg = plsc.load_gather(alls_v, [lane * _L + 2])
    cols = plsc.load_gather(alls_v, [lane * _L + 3])
    beforem = (lane < sid).astype(jnp.float32)
    eqpre_p = jnp.sum(colp * beforem)
    eqpre_n = jnp.sum(coln * beforem)
    qpos = (kp.astype(jnp.float32) - eqpre_p).astype(jnp.int32)
    qneg = (kn.astype(jnp.float32) - eqpre_n).astype(jnp.int32)

    tot_g = jnp.sum(colg)
    tot_s = jnp.sum(cols)
    tval = lax.bitcast_convert_type(pp, jnp.float32)
    fgv = tot_g + kp.astype(jnp.float32) * tval
    sdv = tot_s * jnp.float32(1.0 / n)
    outv = (fgv * (lane == 0).astype(jnp.float32)
            + sdv * (lane == 1).astype(jnp.float32))
    scal_v[...] = outv

    @pl.when(sid == 0)
    def _():
      pltpu.sync_copy(scal_v, scal_hbm)

    def mask_body(i, carry):
      lep, len_ = carry
      v = ts_v[pl.ds(i * _L, _L)]
      bits = lax.bitcast_convert_type(v, jnp.int32)
      eqp = bits == pp
      eqn = bits == pn
      cump = jnp.cumsum(eqp.astype(jnp.int32))
      cumn = jnp.cumsum(eqn.astype(jnp.int32))
      pos_sel = (bits > pp) | (eqp & ((lep + cump) <= qpos))
      neg_sel = (bits < pn) | (eqn & ((len_ + cumn) <= qneg))
      pos_v[pl.ds(i * _L, _L)] = (pos_sel & jnp.logical_not(neg_sel)).astype(jnp.int32)
      neg_v[pl.ds(i * _L, _L)] = neg_sel.astype(jnp.int32)
      return (lep + plsc.all_reduce_population_count(eqp),
              len_ + plsc.all_reduce_population_count(eqn))
    lax.fori_loop(0, nv, mask_body, (zi, zi))

    pltpu.sync_copy(pos_v, pos_hbm.at[pl.ds(base, rows)])
    pltpu.sync_copy(neg_v, neg_hbm.at[pl.ds(base, rows)])

  return sel(ts)


def kernel(t_cls_scores, t_centernesses):
  n, _ = t_cls_scores.shape
  k = max(int(n * 0.01), 2)
  ts, joint = _dense_stage(t_cls_scores, t_centernesses)
  posm, negm, scal = _select_stage(ts, k)
  return (posm > 0, negm > 0, joint, scal[0], scal[1], joint)


# speedup vs baseline: 1.8804x; 1.5549x over previous
import functools

import jax
import jax.numpy as jnp
from jax import lax
from jax.experimental import pallas as pl
from jax.experimental.pallas import tpu as pltpu
from jax.experimental.pallas import tpu_sc as plsc

_L = 16
_NSUB = 16
_BLK = 512


def _sc_stage(cls_scores, cent, k):
  n, nc = cls_scores.shape
  rows = n // _NSUB
  nv = rows // _L
  crows = 496
  nchunk = rows // crows
  gpc = crows // _L

  mesh = plsc.VectorSubcoreMesh(
      core_axis_name="c", subcore_axis_name="s", num_cores=1)

  out_type = (
      jax.ShapeDtypeStruct((n,), jnp.float32),
      jax.ShapeDtypeStruct((n,), jnp.float32),
      jax.ShapeDtypeStruct((n,), jnp.float32),
      jax.ShapeDtypeStruct((_L,), jnp.float32),
  )
  scratch = [
      pltpu.VMEM((crows, nc), jnp.float32),
      pltpu.VMEM((rows,), jnp.float32),
      pltpu.VMEM((rows,), jnp.float32),
      pltpu.VMEM((rows,), jnp.float32),
      pltpu.VMEM((_L * 256,), jnp.int32),
      pltpu.VMEM((_L * 256,), jnp.int32),
      pltpu.VMEM((512,), jnp.int32),
      pltpu.VMEM((_NSUB * 512,), jnp.int32),
      pltpu.VMEM((512,), jnp.int32),
      pltpu.VMEM((_L,), jnp.float32),
      pltpu.VMEM((_NSUB * _L,), jnp.float32),
      pltpu.VMEM((_L,), jnp.float32),
      pltpu.VMEM_SHARED((4, _NSUB * 512), jnp.int32),
      pltpu.VMEM_SHARED((_NSUB * _L,), jnp.float32),
  ]

  @functools.partial(
      pl.kernel, out_type=out_type, mesh=mesh, scratch_types=scratch,
      compiler_params=pltpu.CompilerParams(needs_layout_passes=False))
  def sel(cls_hbm, cent_hbm, joint_hbm, pos_hbm, neg_hbm, scal_hbm,
          cls_v, cent_v, joint_v, ts_v, hp_v, hn_v,
          red_v, allh_v, gh_v, st_v, alls_v, scal_v, shist, sstat):
    pos_v = joint_v
    neg_v = cent_v
    sid = lax.axis_index("s")
    base = sid * rows
    lane = lax.iota(jnp.int32, _L)
    ones = jnp.ones((_L,), jnp.int32)
    zi = jnp.zeros((_L,), jnp.int32)
    zf = jnp.zeros((_L,), jnp.float32)

    pltpu.sync_copy(cent_hbm.at[pl.ds(base, rows)], cent_v)
    cidx = [jnp.full((_L,), c, jnp.int32) for c in range(nc)]

    def chunk_body(ch, _):
      row0 = ch * crows
      pltpu.sync_copy(cls_hbm.at[pl.ds(base + row0, crows), :], cls_v)

      def g_body(g, _):
        ridx = g * _L + lane
        m = plsc.load_gather(cls_v, [ridx, cidx[0]])
        for c in range(1, nc):
          m = jnp.maximum(m, plsc.load_gather(cls_v, [ridx, cidx[c]]))
        s = 1.0 / (1.0 + jnp.exp(-m))
        off = row0 + g * _L
        ts_v[pl.ds(off, _L)] = s
        cv = cent_v[pl.ds(off, _L)]
        joint_v[pl.ds(off, _L)] = (1.0 / (1.0 + jnp.exp(-cv))) * s
        return 0
      lax.fori_loop(0, gpc, g_body, 0)
      return 0
    lax.fori_loop(0, nchunk, chunk_body, 0)
    pltpu.sync_copy(joint_v, joint_hbm.at[pl.ds(base, rows)])

    kp = jnp.int32(k)
    kn = jnp.int32(k)
    pp = jnp.int32(0)
    pn = jnp.int32(0)
    for r in range(4):
      sh = 24 - 8 * r

      def zb(i, _):
        hp_v[pl.ds(i * _L, _L)] = zi
        hn_v[pl.ds(i * _L, _L)] = zi
        return 0
      lax.fori_loop(0, 256, zb, 0)

      if r == 0:
        def sc0(i, _):
          bits = lax.bitcast_convert_type(ts_v[pl.ds(i * _L, _L)], jnp.int32)
          byte = (bits >> sh) & 255
          plsc.addupdate_scatter(hp_v, [lane * 256 + byte], ones)
          return 0
        lax.fori_loop(0, nv, sc0, 0)
      else:
        mh = jnp.int32(-(1 << (sh + 8)))
        pph = pp
        pnh = pn

        def scr(i, _):
          bits = lax.bitcast_convert_type(ts_v[pl.ds(i * _L, _L)], jnp.int32)
          byte = (bits >> sh) & 255
          idx = lane * 256 + byte
          hi = bits & mh
          plsc.addupdate_scatter(hp_v, [idx], ones, mask=(hi == pph))
          plsc.addupdate_scatter(hn_v, [idx], ones, mask=(hi == pnh))
          return 0
        lax.fori_loop(0, nv, scr, 0)

      def lr(j, _):
        accp = zi
        accn = zi
        for l in range(_L):
          accp = accp + hp_v[pl.ds(l * 256 + j * _L, _L)]
          accn = accn + hn_v[pl.ds(l * 256 + j * _L, _L)]
        red_v[pl.ds(j * _L, _L)] = accp
        red_v[pl.ds(256 + j * _L, _L)] = accn
        return 0
      lax.fori_loop(0, 16, lr, 0)

      pltpu.sync_copy(red_v, shist.at[r, pl.ds(sid * 512, 512)])
      plsc.subcore_barrier()
      pltpu.sync_copy(shist.at[r], allh_v)

      def gm(j, _):
        accp = zi
        accn = zi
        for ss in range(_NSUB):
          accp = accp + allh_v[pl.ds(ss * 512 + j * _L, _L)]
          accn = accn + allh_v[pl.ds(ss * 512 + 256 + j * _L, _L)]
        gh_v[pl.ds(j * _L, _L)] = accp
        gh_v[pl.ds(256 + j * _L, _L)] = accn
        return 0
      lax.fori_loop(0, 16, gm, 0)

      noff = 0 if r == 0 else 256

      def htot(off):
        def tb(j, acc):
          return acc + gh_v[pl.ds(off + j * _L, _L)]
        return jnp.sum(lax.fori_loop(0, 16, tb, zi))

      total_p = htot(0)
      total_n = total_p if r == 0 else htot(256)

      def mb_desc(j, carry):
        cnt, hsum, rowpref = carry
        h = gh_v[pl.ds(j * _L, _L)]
        cums = jnp.cumsum(h)
        pref_lt = rowpref + cums - h
        m = (total_p - pref_lt) >= kp
        return (cnt + plsc.all_reduce_population_count(m),
                hsum + jnp.sum(jnp.where(m, h, 0)),
                rowpref + jnp.sum(h))
      cntp, hsump, _ = lax.fori_loop(
          0, 16, mb_desc, (zi, jnp.int32(0), jnp.int32(0)))
      bp = cntp - 1
      abovep = total_p - hsump

      def mb_asc(j, carry):
        cnt, hsum, rowpref = carry
        h = gh_v[pl.ds(noff + j * _L, _L)]
        cums = jnp.cumsum(h)
        m = (rowpref + cums) >= kn
        return (cnt + plsc.all_reduce_population_count(m),
                hsum + jnp.sum(jnp.where(m, h, 0)),
                rowpref + jnp.sum(h))
      cntn, hsumn, _ = lax.fori_loop(
          0, 16, mb_asc, (zi, jnp.int32(0), jnp.int32(0)))
      bn = 256 - cntn
      belown = total_n - hsumn

      kp = kp - abovep
      pp = pp | (bp << sh)
      kn = kn - belown
      pn = pn | (bn << sh)

    def cnt_body(i, carry):
      cp, cn, sg, st = carry
      v = ts_v[pl.ds(i * _L, _L)]
      bits = lax.bitcast_convert_type(v, jnp.int32)
      cp = cp + (bits == pp).astype(jnp.int32)
      cn = cn + (bits == pn).astype(jnp.int32)
      sg = sg + jnp.where(bits > pp, v, 0.0)
      st = st + v
      return (cp, cn, sg, st)
    cpv, cnv, sgv, stv = lax.fori_loop(0, nv, cnt_body, (zi, zi, zf, zf))
    cposf = jnp.sum(cpv).astype(jnp.float32)
    cnegf = jnp.sum(cnv).astype(jnp.float32)
    sgt = jnp.sum(sgv)
    stot = jnp.sum(stv)

    stats = (cposf * (lane == 0).astype(jnp.float32)
             + cnegf * (lane == 1).astype(jnp.float32)
             + sgt * (lane == 2).astype(jnp.float32)
             + stot * (lane == 3).astype(jnp.float32))
    st_v[...] = stats
    pltpu.sync_copy(st_v, sstat.at[pl.ds(sid * _L, _L)])
    plsc.subcore_barrier()
    pltpu.sync_copy(sstat, alls_v)

    colp = plsc.load_gather(alls_v, [lane * _L])
    coln = plsc.load_gather(alls_v, [lane * _L + 1])
    colg = plsc.load_gather(alls_v, [lane * _L + 2])
    cols = plsc.load_gather(alls_v, [lane * _L + 3])
    beforem = (lane < sid).astype(jnp.float32)
    eqpre_p = jnp.sum(colp * beforem)
    eqpre_n = jnp.sum(coln * beforem)
    qpos = (kp.astype(jnp.float32) - eqpre_p).astype(jnp.int32)
    qneg = (kn.astype(jnp.float32) - eqpre_n).astype(jnp.int32)

    tot_g = jnp.sum(colg)
    tot_s = jnp.sum(cols)
    tval = lax.bitcast_convert_type(pp, jnp.float32)
    fgv = tot_g + kp.astype(jnp.float32) * tval
    sdv = tot_s * jnp.float32(1.0 / n)
    outv = (fgv * (lane == 0).astype(jnp.float32)
            + sdv * (lane == 1).astype(jnp.float32))
    scal_v[...] = outv

    @pl.when(sid == 0)
    def _():
      pltpu.sync_copy(scal_v, scal_hbm)

    def mask_body(i, carry):
      lep, len_ = carry
      v = ts_v[pl.ds(i * _L, _L)]
      bits = lax.bitcast_convert_type(v, jnp.int32)
      eqp = bits == pp
      eqn = bits == pn
      cump = jnp.cumsum(eqp.astype(jnp.int32))
      cumn = jnp.cumsum(eqn.astype(jnp.int32))
      pos_sel = (bits > pp) | (eqp & ((lep + cump) <= qpos))
      neg_sel = (bits < pn) | (eqn & ((len_ + cumn) <= qneg))
      pos_v[pl.ds(i * _L, _L)] = (pos_sel & jnp.logical_not(neg_sel)).astype(jnp.float32)
      neg_v[pl.ds(i * _L, _L)] = neg_sel.astype(jnp.float32)
      return (lep + plsc.all_reduce_population_count(eqp),
              len_ + plsc.all_reduce_population_count(eqn))
    lax.fori_loop(0, nv, mask_body, (zi, zi))

    pltpu.sync_copy(pos_v, pos_hbm.at[pl.ds(base, rows)])
    pltpu.sync_copy(neg_v, neg_hbm.at[pl.ds(base, rows)])

  return sel(cls_scores, cent)


def kernel(t_cls_scores, t_centernesses):
  n, _ = t_cls_scores.shape
  k = max(int(n * 0.01), 2)
  joint, posm, negm, scal = _sc_stage(t_cls_scores, t_centernesses.reshape(-1), k)
  return (posm > 0, negm > 0, joint, scal[0], scal[1], joint)


# speedup vs baseline: 2.3956x; 1.2740x over previous
import functools

import jax
import jax.numpy as jnp
from jax import lax
from jax.experimental import pallas as pl
from jax.experimental.pallas import tpu as pltpu
from jax.experimental.pallas import tpu_sc as plsc

_L = 16
_NSUB = 16
_BLK = 512


def _sc_stage(cls_scores, cent, k):
  n, nc = cls_scores.shape
  rows = n // _NSUB
  nv = rows // _L
  crows = 176
  nchunk = rows // crows
  gpc = crows // _L

  mesh = plsc.VectorSubcoreMesh(
      core_axis_name="c", subcore_axis_name="s", num_cores=1)

  out_type = (
      jax.ShapeDtypeStruct((n,), jnp.float32),
      jax.ShapeDtypeStruct((n,), jnp.float32),
      jax.ShapeDtypeStruct((n,), jnp.float32),
      jax.ShapeDtypeStruct((_L,), jnp.float32),
  )
  scratch = [
      pltpu.VMEM((crows, nc), jnp.float32),
      pltpu.VMEM((crows, nc), jnp.float32),
      pltpu.VMEM((rows,), jnp.float32),
      pltpu.VMEM((rows,), jnp.float32),
      pltpu.VMEM((rows,), jnp.float32),
      pltpu.VMEM((_L * 256,), jnp.int32),
      pltpu.VMEM((_L * 256,), jnp.int32),
      pltpu.VMEM((512,), jnp.int32),
      pltpu.VMEM((_NSUB * 512,), jnp.int32),
      pltpu.VMEM((512,), jnp.int32),
      pltpu.VMEM((_L,), jnp.float32),
      pltpu.VMEM((_NSUB * _L,), jnp.float32),
      pltpu.VMEM((_L,), jnp.float32),
      pltpu.VMEM_SHARED((4, _NSUB * 512), jnp.int32),
      pltpu.VMEM_SHARED((_NSUB * _L,), jnp.float32),
      pltpu.SemaphoreType.DMA,
      pltpu.SemaphoreType.DMA,
  ]

  @functools.partial(
      pl.kernel, out_type=out_type, mesh=mesh, scratch_types=scratch,
      compiler_params=pltpu.CompilerParams(needs_layout_passes=False))
  def sel(cls_hbm, cent_hbm, joint_hbm, pos_hbm, neg_hbm, scal_hbm,
          cls_a, cls_b, cent_v, joint_v, ts_v, hp_v, hn_v,
          red_v, allh_v, gh_v, st_v, alls_v, scal_v, shist, sstat,
          sem_a, sem_b):
    pos_v = joint_v
    neg_v = cent_v
    sid = lax.axis_index("s")
    base = sid * rows
    lane = lax.iota(jnp.int32, _L)
    ones = jnp.ones((_L,), jnp.int32)
    zi = jnp.zeros((_L,), jnp.int32)
    zf = jnp.zeros((_L,), jnp.float32)

    pltpu.sync_copy(cent_hbm.at[pl.ds(base, rows)], cent_v)
    cidx = [jnp.full((_L,), c, jnp.int32) for c in range(nc)]

    def cls_dma(ch, buf, sem):
      return pltpu.make_async_copy(
          cls_hbm.at[pl.ds(base + ch * crows, crows), :], buf, sem)

    def rowmax_chunk(ch, buf):
      row0 = ch * crows

      def g_body(g, _):
        ridx = g * _L + lane
        m = plsc.load_gather(buf, [ridx, cidx[0]])
        for c in range(1, nc):
          m = jnp.maximum(m, plsc.load_gather(buf, [ridx, cidx[c]]))
        ts_v[pl.ds(row0 + g * _L, _L)] = m
        return 0
      lax.fori_loop(0, gpc, g_body, 0)

    nsuper = nchunk // 2
    cls_dma(0, cls_a, sem_a).start()

    def super_body(s2, _):
      ch0 = s2 * 2
      cls_dma(ch0, cls_a, sem_a).wait()
      cls_dma(ch0 + 1, cls_b, sem_b).start()
      rowmax_chunk(ch0, cls_a)
      cls_dma(ch0 + 1, cls_b, sem_b).wait()

      @pl.when(s2 < nsuper - 1)
      def _():
        cls_dma(ch0 + 2, cls_a, sem_a).start()
      rowmax_chunk(ch0 + 1, cls_b)
      return 0
    lax.fori_loop(0, nsuper, super_body, 0)

    def sig_body(i, _):
      o = i * (2 * _L)
      m0 = ts_v[pl.ds(o, _L)]
      m1 = ts_v[pl.ds(o + _L, _L)]
      c0 = cent_v[pl.ds(o, _L)]
      c1 = cent_v[pl.ds(o + _L, _L)]
      s0 = 1.0 / (1.0 + jnp.exp(-m0))
      s1 = 1.0 / (1.0 + jnp.exp(-m1))
      e0 = 1.0 / (1.0 + jnp.exp(-c0))
      e1 = 1.0 / (1.0 + jnp.exp(-c1))
      ts_v[pl.ds(o, _L)] = s0
      ts_v[pl.ds(o + _L, _L)] = s1
      joint_v[pl.ds(o, _L)] = e0 * s0
      joint_v[pl.ds(o + _L, _L)] = e1 * s1
      return 0
    lax.fori_loop(0, nv // 2, sig_body, 0)
    pltpu.sync_copy(joint_v, joint_hbm.at[pl.ds(base, rows)])

    kp = jnp.int32(k)
    kn = jnp.int32(k)
    pp = jnp.int32(0)
    pn = jnp.int32(0)
    for r in range(4):
      sh = 24 - 8 * r

      def zb(i, _):
        hp_v[pl.ds(i * _L, _L)] = zi
        hn_v[pl.ds(i * _L, _L)] = zi
        return 0
      lax.fori_loop(0, 256, zb, 0)

      if r == 0:
        def sc0(i, _):
          bits = lax.bitcast_convert_type(ts_v[pl.ds(i * _L, _L)], jnp.int32)
          byte = (bits >> sh) & 255
          plsc.addupdate_scatter(hp_v, [lane * 256 + byte], ones)
          return 0
        lax.fori_loop(0, nv, sc0, 0)
      else:
        mh = jnp.int32(-(1 << (sh + 8)))
        pph = pp
        pnh = pn

        def scr(i, _):
          bits = lax.bitcast_convert_type(ts_v[pl.ds(i * _L, _L)], jnp.int32)
          byte = (bits >> sh) & 255
          idx = lane * 256 + byte
          hi = bits & mh
          plsc.addupdate_scatter(hp_v, [idx], ones, mask=(hi == pph))
          plsc.addupdate_scatter(hn_v, [idx], ones, mask=(hi == pnh))
          return 0
        lax.fori_loop(0, nv, scr, 0)

      def lr(j, _):
        accp = zi
        accn = zi
        for l in range(_L):
          accp = accp + hp_v[pl.ds(l * 256 + j * _L, _L)]
          accn = accn + hn_v[pl.ds(l * 256 + j * _L, _L)]
        red_v[pl.ds(j * _L, _L)] = accp
        red_v[pl.ds(256 + j * _L, _L)] = accn
        return 0
      lax.fori_loop(0, 16, lr, 0)

      pltpu.sync_copy(red_v, shist.at[r, pl.ds(sid * 512, 512)])
      plsc.subcore_barrier()
      pltpu.sync_copy(shist.at[r], allh_v)

      def gm(j, _):
        accp = zi
        accn = zi
        for ss in range(_NSUB):
          accp = accp + allh_v[pl.ds(ss * 512 + j * _L, _L)]
          accn = accn + allh_v[pl.ds(ss * 512 + 256 + j * _L, _L)]
        gh_v[pl.ds(j * _L, _L)] = accp
        gh_v[pl.ds(256 + j * _L, _L)] = accn
        return 0
      lax.fori_loop(0, 16, gm, 0)

      noff = 0 if r == 0 else 256

      def htot(off):
        def tb(j, acc):
          return acc + gh_v[pl.ds(off + j * _L, _L)]
        return jnp.sum(lax.fori_loop(0, 16, tb, zi))

      total_p = htot(0)
      total_n = total_p if r == 0 else htot(256)

      def mb_desc(j, carry):
        cnt, hsum, rowpref = carry
        h = gh_v[pl.ds(j * _L, _L)]
        cums = jnp.cumsum(h)
        pref_lt = rowpref + cums - h
        m = (total_p - pref_lt) >= kp
        return (cnt + plsc.all_reduce_population_count(m),
                hsum + jnp.sum(jnp.where(m, h, 0)),
                rowpref + jnp.sum(h))
      cntp, hsump, _ = lax.fori_loop(
          0, 16, mb_desc, (zi, jnp.int32(0), jnp.int32(0)))
      bp = cntp - 1
      abovep = total_p - hsump

      def mb_asc(j, carry):
        cnt, hsum, rowpref = carry
        h = gh_v[pl.ds(noff + j * _L, _L)]
        cums = jnp.cumsum(h)
        m = (rowpref + cums) >= kn
        return (cnt + plsc.all_reduce_population_count(m),
                hsum + jnp.sum(jnp.where(m, h, 0)),
                rowpref + jnp.sum(h))
      cntn, hsumn, _ = lax.fori_loop(
          0, 16, mb_asc, (zi, jnp.int32(0), jnp.int32(0)))
      bn = 256 - cntn
      belown = total_n - hsumn

      kp = kp - abovep
      pp = pp | (bp << sh)
      kn = kn - belown
      pn = pn | (bn << sh)

    def cnt_body(i, carry):
      cp, cn, sg, st = carry
      v = ts_v[pl.ds(i * _L, _L)]
      bits = lax.bitcast_convert_type(v, jnp.int32)
      cp = cp + (bits == pp).astype(jnp.int32)
      cn = cn + (bits == pn).astype(jnp.int32)
      sg = sg + jnp.where(bits > pp, v, 0.0)
      st = st + v
      return (cp, cn, sg, st)
    cpv, cnv, sgv, stv = lax.fori_loop(0, nv, cnt_body, (zi, zi, zf, zf))
    cposf = jnp.sum(cpv).astype(jnp.float32)
    cnegf = jnp.sum(cnv).astype(jnp.float32)
    sgt = jnp.sum(sgv)
    stot = jnp.sum(stv)

    stats = (cposf * (lane == 0).astype(jnp.float32)
             + cnegf * (lane == 1).astype(jnp.float32)
             + sgt * (lane == 2).astype(jnp.float32)
             + stot * (lane == 3).astype(jnp.float32))
    st_v[...] = stats
    pltpu.sync_copy(st_v, sstat.at[pl.ds(sid * _L, _L)])
    plsc.subcore_barrier()
    pltpu.sync_copy(sstat, alls_v)

    colp = plsc.load_gather(alls_v, [lane * _L])
    coln = plsc.load_gather(alls_v, [lane * _L + 1])
    colg = plsc.load_gather(alls_v, [lane * _L + 2])
    cols = plsc.load_gather(alls_v, [lane * _L + 3])
    beforem = (lane < sid).astype(jnp.float32)
    eqpre_p = jnp.sum(colp * beforem)
    eqpre_n = jnp.sum(coln * beforem)
    qpos = (kp.astype(jnp.float32) - eqpre_p).astype(jnp.int32)
    qneg = (kn.astype(jnp.float32) - eqpre_n).astype(jnp.int32)

    tot_g = jnp.sum(colg)
    tot_s = jnp.sum(cols)
    tval = lax.bitcast_convert_type(pp, jnp.float32)
    fgv = tot_g + kp.astype(jnp.float32) * tval
    sdv = tot_s * jnp.float32(1.0 / n)
    outv = (fgv * (lane == 0).astype(jnp.float32)
            + sdv * (lane == 1).astype(jnp.float32))
    scal_v[...] = outv

    @pl.when(sid == 0)
    def _():
      pltpu.sync_copy(scal_v, scal_hbm)

    def mask_body(i, carry):
      lep, len_ = carry
      v = ts_v[pl.ds(i * _L, _L)]
      bits = lax.bitcast_convert_type(v, jnp.int32)
      eqp = bits == pp
      eqn = bits == pn
      cump = jnp.cumsum(eqp.astype(jnp.int32))
      cumn = jnp.cumsum(eqn.astype(jnp.int32))
      pos_sel = (bits > pp) | (eqp & ((lep + cump) <= qpos))
      neg_sel = (bits < pn) | (eqn & ((len_ + cumn) <= qneg))
      pos_v[pl.ds(i * _L, _L)] = (pos_sel & jnp.logical_not(neg_sel)).astype(jnp.float32)
      neg_v[pl.ds(i * _L, _L)] = neg_sel.astype(jnp.float32)
      return (lep + plsc.all_reduce_population_count(eqp),
              len_ + plsc.all_reduce_population_count(eqn))
    lax.fori_loop(0, nv, mask_body, (zi, zi))

    pltpu.sync_copy(pos_v, pos_hbm.at[pl.ds(base, rows)])
    pltpu.sync_copy(neg_v, neg_hbm.at[pl.ds(base, rows)])

  return sel(cls_scores, cent)


def kernel(t_cls_scores, t_centernesses):
  n, _ = t_cls_scores.shape
  k = max(int(n * 0.01), 2)
  joint, posm, negm, scal = _sc_stage(t_cls_scores, t_centernesses.reshape(-1), k)
  return (posm > 0, negm > 0, joint, scal[0], scal[1], joint)


# speedup vs baseline: 2.7477x; 1.1470x over previous
import functools

import jax
import jax.numpy as jnp
from jax import lax
from jax.experimental import pallas as pl
from jax.experimental.pallas import tpu as pltpu
from jax.experimental.pallas import tpu_sc as plsc

_L = 16
_NSUB = 16
_BLK = 512


def _sc_stage(cls_flat, cent, k, nc):
  n = cls_flat.shape[0] // nc
  rows = n // _NSUB
  nv = rows // _L
  crows = 176
  nchunk = rows // crows
  gpc = crows // _L

  mesh = plsc.VectorSubcoreMesh(
      core_axis_name="c", subcore_axis_name="s", num_cores=1)

  out_type = (
      jax.ShapeDtypeStruct((n,), jnp.float32),
      jax.ShapeDtypeStruct((n,), jnp.float32),
      jax.ShapeDtypeStruct((n,), jnp.float32),
      jax.ShapeDtypeStruct((_L,), jnp.float32),
  )
  scratch = [
      pltpu.VMEM((crows * nc,), jnp.float32),
      pltpu.VMEM((crows * nc,), jnp.float32),
      pltpu.VMEM((rows,), jnp.float32),
      pltpu.VMEM((rows,), jnp.float32),
      pltpu.VMEM((rows,), jnp.float32),
      pltpu.VMEM((_L * 256,), jnp.int32),
      pltpu.VMEM((_L * 256,), jnp.int32),
      pltpu.VMEM((512,), jnp.int32),
      pltpu.VMEM((_NSUB * 512,), jnp.int32),
      pltpu.VMEM((512,), jnp.int32),
      pltpu.VMEM((_L,), jnp.float32),
      pltpu.VMEM((_NSUB * _L,), jnp.float32),
      pltpu.VMEM((_L,), jnp.float32),
      pltpu.VMEM_SHARED((4, _NSUB * 512), jnp.int32),
      pltpu.VMEM_SHARED((_NSUB * _L,), jnp.float32),
      pltpu.SemaphoreType.DMA,
      pltpu.SemaphoreType.DMA,
  ]

  @functools.partial(
      pl.kernel, out_type=out_type, mesh=mesh, scratch_types=scratch,
      compiler_params=pltpu.CompilerParams(needs_layout_passes=False))
  def sel(cls_hbm, cent_hbm, joint_hbm, pos_hbm, neg_hbm, scal_hbm,
          cls_a, cls_b, cent_v, joint_v, ts_v, hp_v, hn_v,
          red_v, allh_v, gh_v, st_v, alls_v, scal_v, shist, sstat,
          sem_a, sem_b):
    pos_v = joint_v
    neg_v = cent_v
    sid = lax.axis_index("s")
    base = sid * rows
    lane = lax.iota(jnp.int32, _L)
    ones = jnp.ones((_L,), jnp.int32)
    zi = jnp.zeros((_L,), jnp.int32)
    zf = jnp.zeros((_L,), jnp.float32)

    pltpu.sync_copy(cent_hbm.at[pl.ds(base, rows)], cent_v)
    cidx = [jnp.full((_L,), c, jnp.int32) for c in range(nc)]

    def cls_dma(ch, buf, sem):
      return pltpu.make_async_copy(
          cls_hbm.at[pl.ds((base + ch * crows) * nc, crows * nc)], buf, sem)

    def rowmax_chunk(ch, buf):
      row0 = ch * crows

      def g_body(g, _):
        ridx = (g * _L + lane) * nc
        m = plsc.load_gather(buf, [ridx + cidx[0]])
        for c in range(1, nc):
          m = jnp.maximum(m, plsc.load_gather(buf, [ridx + cidx[c]]))
        ts_v[pl.ds(row0 + g * _L, _L)] = m
        return 0
      lax.fori_loop(0, gpc, g_body, 0)

    nsuper = nchunk // 2
    cls_dma(0, cls_a, sem_a).start()

    def super_body(s2, _):
      ch0 = s2 * 2
      cls_dma(ch0, cls_a, sem_a).wait()
      cls_dma(ch0 + 1, cls_b, sem_b).start()
      rowmax_chunk(ch0, cls_a)
      cls_dma(ch0 + 1, cls_b, sem_b).wait()

      @pl.when(s2 < nsuper - 1)
      def _():
        cls_dma(ch0 + 2, cls_a, sem_a).start()
      rowmax_chunk(ch0 + 1, cls_b)
      return 0
    lax.fori_loop(0, nsuper, super_body, 0)

    def sig_body(i, _):
      o = i * (2 * _L)
      m0 = ts_v[pl.ds(o, _L)]
      m1 = ts_v[pl.ds(o + _L, _L)]
      c0 = cent_v[pl.ds(o, _L)]
      c1 = cent_v[pl.ds(o + _L, _L)]
      s0 = 1.0 / (1.0 + jnp.exp(-m0))
      s1 = 1.0 / (1.0 + jnp.exp(-m1))
      e0 = 1.0 / (1.0 + jnp.exp(-c0))
      e1 = 1.0 / (1.0 + jnp.exp(-c1))
      ts_v[pl.ds(o, _L)] = s0
      ts_v[pl.ds(o + _L, _L)] = s1
      joint_v[pl.ds(o, _L)] = e0 * s0
      joint_v[pl.ds(o + _L, _L)] = e1 * s1
      return 0
    lax.fori_loop(0, nv // 2, sig_body, 0)
    pltpu.sync_copy(joint_v, joint_hbm.at[pl.ds(base, rows)])

    kp = jnp.int32(k)
    kn = jnp.int32(k)
    pp = jnp.int32(0)
    pn = jnp.int32(0)
    for r in range(4):
      sh = 24 - 8 * r

      def zb(i, _):
        hp_v[pl.ds(i * _L, _L)] = zi
        hn_v[pl.ds(i * _L, _L)] = zi
        return 0
      lax.fori_loop(0, 256, zb, 0)

      if r == 0:
        def sc0(i, _):
          bits = lax.bitcast_convert_type(ts_v[pl.ds(i * _L, _L)], jnp.int32)
          byte = (bits >> sh) & 255
          plsc.addupdate_scatter(hp_v, [lane * 256 + byte], ones)
          return 0
        lax.fori_loop(0, nv, sc0, 0)
      else:
        mh = jnp.int32(-(1 << (sh + 8)))
        pph = pp
        pnh = pn

        def scr(i, _):
          bits = lax.bitcast_convert_type(ts_v[pl.ds(i * _L, _L)], jnp.int32)
          byte = (bits >> sh) & 255
          idx = lane * 256 + byte
          hi = bits & mh
          plsc.addupdate_scatter(hp_v, [idx], ones, mask=(hi == pph))
          plsc.addupdate_scatter(hn_v, [idx], ones, mask=(hi == pnh))
          return 0
        lax.fori_loop(0, nv, scr, 0)

      def lr(j, _):
        accp = zi
        accn = zi
        for l in range(_L):
          accp = accp + hp_v[pl.ds(l * 256 + j * _L, _L)]
          accn = accn + hn_v[pl.ds(l * 256 + j * _L, _L)]
        red_v[pl.ds(j * _L, _L)] = accp
        red_v[pl.ds(256 + j * _L, _L)] = accn
        return 0
      lax.fori_loop(0, 16, lr, 0)

      pltpu.sync_copy(red_v, shist.at[r, pl.ds(sid * 512, 512)])
      plsc.subcore_barrier()
      pltpu.sync_copy(shist.at[r], allh_v)

      def gm(j, _):
        accp = zi
        accn = zi
        for ss in range(_NSUB):
          accp = accp + allh_v[pl.ds(ss * 512 + j * _L, _L)]
          accn = accn + allh_v[pl.ds(ss * 512 + 256 + j * _L, _L)]
        gh_v[pl.ds(j * _L, _L)] = accp
        gh_v[pl.ds(256 + j * _L, _L)] = accn
        return 0
      lax.fori_loop(0, 16, gm, 0)

      noff = 0 if r == 0 else 256

      def htot(off):
        def tb(j, acc):
          return acc + gh_v[pl.ds(off + j * _L, _L)]
        return jnp.sum(lax.fori_loop(0, 16, tb, zi))

      total_p = htot(0)
      total_n = total_p if r == 0 else htot(256)

      def mb_desc(j, carry):
        cnt, hsum, rowpref = carry
        h = gh_v[pl.ds(j * _L, _L)]
        cums = jnp.cumsum(h)
        pref_lt = rowpref + cums - h
        m = (total_p - pref_lt) >= kp
        return (cnt + plsc.all_reduce_population_count(m),
                hsum + jnp.sum(jnp.where(m, h, 0)),
                rowpref + jnp.sum(h))
      cntp, hsump, _ = lax.fori_loop(
          0, 16, mb_desc, (zi, jnp.int32(0), jnp.int32(0)))
      bp = cntp - 1
      abovep = total_p - hsump

      def mb_asc(j, carry):
        cnt, hsum, rowpref = carry
        h = gh_v[pl.ds(noff + j * _L, _L)]
        cums = jnp.cumsum(h)
        m = (rowpref + cums) >= kn
        return (cnt + plsc.all_reduce_population_count(m),
                hsum + jnp.sum(jnp.where(m, h, 0)),
                rowpref + jnp.sum(h))
      cntn, hsumn, _ = lax.fori_loop(
          0, 16, mb_asc, (zi, jnp.int32(0), jnp.int32(0)))
      bn = 256 - cntn
      belown = total_n - hsumn

      kp = kp - abovep
      pp = pp | (bp << sh)
      kn = kn - belown
      pn = pn | (bn << sh)

    def cnt_body(i, carry):
      cp, cn, sg, st = carry
      v = ts_v[pl.ds(i * _L, _L)]
      bits = lax.bitcast_convert_type(v, jnp.int32)
      cp = cp + (bits == pp).astype(jnp.int32)
      cn = cn + (bits == pn).astype(jnp.int32)
      sg = sg + jnp.where(bits > pp, v, 0.0)
      st = st + v
      return (cp, cn, sg, st)
    cpv, cnv, sgv, stv = lax.fori_loop(0, nv, cnt_body, (zi, zi, zf, zf))
    cposf = jnp.sum(cpv).astype(jnp.float32)
    cnegf = jnp.sum(cnv).astype(jnp.float32)
    sgt = jnp.sum(sgv)
    stot = jnp.sum(stv)

    stats = (cposf * (lane == 0).astype(jnp.float32)
             + cnegf * (lane == 1).astype(jnp.float32)
             + sgt * (lane == 2).astype(jnp.float32)
             + stot * (lane == 3).astype(jnp.float32))
    st_v[...] = stats
    pltpu.sync_copy(st_v, sstat.at[pl.ds(sid * _L, _L)])
    plsc.subcore_barrier()
    pltpu.sync_copy(sstat, alls_v)

    colp = plsc.load_gather(alls_v, [lane * _L])
    coln = plsc.load_gather(alls_v, [lane * _L + 1])
    colg = plsc.load_gather(alls_v, [lane * _L + 2])
    cols = plsc.load_gather(alls_v, [lane * _L + 3])
    beforem = (lane < sid).astype(jnp.float32)
    eqpre_p = jnp.sum(colp * beforem)
    eqpre_n = jnp.sum(coln * beforem)
    qpos = (kp.astype(jnp.float32) - eqpre_p).astype(jnp.int32)
    qneg = (kn.astype(jnp.float32) - eqpre_n).astype(jnp.int32)

    tot_g = jnp.sum(colg)
    tot_s = jnp.sum(cols)
    tval = lax.bitcast_convert_type(pp, jnp.float32)
    fgv = tot_g + kp.astype(jnp.float32) * tval
    sdv = tot_s * jnp.float32(1.0 / n)
    outv = (fgv * (lane == 0).astype(jnp.float32)
            + sdv * (lane == 1).astype(jnp.float32))
    scal_v[...] = outv

    @pl.when(sid == 0)
    def _():
      pltpu.sync_copy(scal_v, scal_hbm)

    def mask_body(i, carry):
      lep, len_ = carry
      v = ts_v[pl.ds(i * _L, _L)]
      bits = lax.bitcast_convert_type(v, jnp.int32)
      eqp = bits == pp
      eqn = bits == pn
      cump = jnp.cumsum(eqp.astype(jnp.int32))
      cumn = jnp.cumsum(eqn.astype(jnp.int32))
      pos_sel = (bits > pp) | (eqp & ((lep + cump) <= qpos))
      neg_sel = (bits < pn) | (eqn & ((len_ + cumn) <= qneg))
      pos_v[pl.ds(i * _L, _L)] = (pos_sel & jnp.logical_not(neg_sel)).astype(jnp.float32)
      neg_v[pl.ds(i * _L, _L)] = neg_sel.astype(jnp.float32)
      return (lep + plsc.all_reduce_population_count(eqp),
              len_ + plsc.all_reduce_population_count(eqn))
    lax.fori_loop(0, nv, mask_body, (zi, zi))

    pltpu.sync_copy(pos_v, pos_hbm.at[pl.ds(base, rows)])
    pltpu.sync_copy(neg_v, neg_hbm.at[pl.ds(base, rows)])

  return sel(cls_flat, cent)


def kernel(t_cls_scores, t_centernesses):
  n, nc = t_cls_scores.shape
  k = max(int(n * 0.01), 2)
  joint, posm, negm, scal = _sc_stage(
      t_cls_scores.reshape(-1), t_centernesses.reshape(-1), k, nc)
  return (posm > 0, negm > 0, joint, scal[0], scal[1], joint)


# speedup vs baseline: 3.0435x; 1.1077x over previous
import functools

import jax
import jax.numpy as jnp
from jax import lax
from jax.experimental import pallas as pl
from jax.experimental.pallas import tpu as pltpu
from jax.experimental.pallas import tpu_sc as plsc

_L = 16
_NSUB = 16
_BLK = 512


def _sc_stage(cls_flat, cent, k, nc):
  n = cls_flat.shape[0] // nc
  rows = n // _NSUB
  nv = rows // _L
  crows = 176
  nchunk = rows // crows
  gpc = crows // _L

  mesh = plsc.VectorSubcoreMesh(
      core_axis_name="c", subcore_axis_name="s", num_cores=1)

  out_type = (
      jax.ShapeDtypeStruct((n,), jnp.float32),
      jax.ShapeDtypeStruct((n,), jnp.float32),
      jax.ShapeDtypeStruct((n,), jnp.float32),
      jax.ShapeDtypeStruct((_L,), jnp.float32),
  )
  scratch = [
      pltpu.VMEM((crows * nc,), jnp.float32),
      pltpu.VMEM((crows * nc,), jnp.float32),
      pltpu.VMEM((crows * nc,), jnp.float32),
      pltpu.VMEM((crows * nc,), jnp.float32),
      pltpu.VMEM((rows,), jnp.float32),
      pltpu.VMEM((rows,), jnp.float32),
      pltpu.VMEM((rows,), jnp.float32),
      pltpu.VMEM((_L * 256,), jnp.int32),
      pltpu.VMEM((_L * 256,), jnp.int32),
      pltpu.VMEM((512,), jnp.int32),
      pltpu.VMEM((_NSUB * 512,), jnp.int32),
      pltpu.VMEM((512,), jnp.int32),
      pltpu.VMEM((_L,), jnp.float32),
      pltpu.VMEM((_NSUB * _L,), jnp.float32),
      pltpu.VMEM((_L,), jnp.float32),
      pltpu.VMEM_SHARED((4, _NSUB * 512), jnp.int32),
      pltpu.VMEM_SHARED((_NSUB * _L,), jnp.float32),
      pltpu.SemaphoreType.DMA,
      pltpu.SemaphoreType.DMA,
      pltpu.SemaphoreType.DMA,
      pltpu.SemaphoreType.DMA,
  ]

  @functools.partial(
      pl.kernel, out_type=out_type, mesh=mesh, scratch_types=scratch,
      compiler_params=pltpu.CompilerParams(needs_layout_passes=False))
  def sel(cls_hbm, cent_hbm, joint_hbm, pos_hbm, neg_hbm, scal_hbm,
          cls_a, cls_b, cls_c, cls_d, cent_v, joint_v, ts_v, hp_v, hn_v,
          red_v, allh_v, gh_v, st_v, alls_v, scal_v, shist, sstat,
          sem_a, sem_b, sem_c, sem_d):
    pos_v = joint_v
    neg_v = cent_v
    sid = lax.axis_index("s")
    base = sid * rows
    lane = lax.iota(jnp.int32, _L)
    ones = jnp.ones((_L,), jnp.int32)
    zi = jnp.zeros((_L,), jnp.int32)
    zf = jnp.zeros((_L,), jnp.float32)

    pltpu.sync_copy(cent_hbm.at[pl.ds(base, rows)], cent_v)
    cidx = [jnp.full((_L,), c, jnp.int32) for c in range(nc)]

    def cls_dma(ch, buf, sem):
      return pltpu.make_async_copy(
          cls_hbm.at[pl.ds((base + ch * crows) * nc, crows * nc)], buf, sem)

    def rowmax_chunk(ch, buf):
      row0 = ch * crows

      def g_body(g, _):
        ridx = (g * _L + lane) * nc
        m = plsc.load_gather(buf, [ridx + cidx[0]])
        for c in range(1, nc):
          m = jnp.maximum(m, plsc.load_gather(buf, [ridx + cidx[c]]))
        ts_v[pl.ds(row0 + g * _L, _L)] = m
        return 0
      lax.fori_loop(0, gpc, g_body, 0)

    bufs = [(cls_a, sem_a), (cls_b, sem_b), (cls_c, sem_c), (cls_d, sem_d)]
    nsuper = (nchunk + 3) // 4
    for ch in range(3):
      cls_dma(ch, *bufs[ch]).start()

    def super_body(i, _):
      for b in range(4):
        ch = i * 4 + b
        buf, sem = bufs[b]

        @pl.when(ch + 3 < nchunk)
        def _():
          nbuf, nsem = bufs[(b + 3) % 4]
          cls_dma(ch + 3, nbuf, nsem).start()

        @pl.when(ch < nchunk)
        def _():
          cls_dma(ch, buf, sem).wait()
          rowmax_chunk(ch, buf)
      return 0
    lax.fori_loop(0, nsuper, super_body, 0)

    def sig_body(i, _):
      o = i * (2 * _L)
      m0 = ts_v[pl.ds(o, _L)]
      m1 = ts_v[pl.ds(o + _L, _L)]
      c0 = cent_v[pl.ds(o, _L)]
      c1 = cent_v[pl.ds(o + _L, _L)]
      s0 = 1.0 / (1.0 + jnp.exp(-m0))
      s1 = 1.0 / (1.0 + jnp.exp(-m1))
      e0 = 1.0 / (1.0 + jnp.exp(-c0))
      e1 = 1.0 / (1.0 + jnp.exp(-c1))
      ts_v[pl.ds(o, _L)] = s0
      ts_v[pl.ds(o + _L, _L)] = s1
      joint_v[pl.ds(o, _L)] = e0 * s0
      joint_v[pl.ds(o + _L, _L)] = e1 * s1
      return 0
    lax.fori_loop(0, nv // 2, sig_body, 0)
    pltpu.sync_copy(joint_v, joint_hbm.at[pl.ds(base, rows)])

    kp = jnp.int32(k)
    kn = jnp.int32(k)
    pp = jnp.int32(0)
    pn = jnp.int32(0)
    for r in range(4):
      sh = 24 - 8 * r

      def zb(i, _):
        hp_v[pl.ds(i * _L, _L)] = zi
        hn_v[pl.ds(i * _L, _L)] = zi
        return 0
      lax.fori_loop(0, 256, zb, 0)

      if r == 0:
        def sc0(i, _):
          bits = lax.bitcast_convert_type(ts_v[pl.ds(i * _L, _L)], jnp.int32)
          byte = (bits >> sh) & 255
          plsc.addupdate_scatter(hp_v, [lane * 256 + byte], ones)
          return 0
        lax.fori_loop(0, nv, sc0, 0)
      else:
        mh = jnp.int32(-(1 << (sh + 8)))
        pph = pp
        pnh = pn

        def scr(i, _):
          bits = lax.bitcast_convert_type(ts_v[pl.ds(i * _L, _L)], jnp.int32)
          byte = (bits >> sh) & 255
          idx = lane * 256 + byte
          hi = bits & mh
          plsc.addupdate_scatter(hp_v, [idx], ones, mask=(hi == pph))
          plsc.addupdate_scatter(hn_v, [idx], ones, mask=(hi == pnh))
          return 0
        lax.fori_loop(0, nv, scr, 0)

      def lr(j, _):
        accp = zi
        accn = zi
        for l in range(_L):
          accp = accp + hp_v[pl.ds(l * 256 + j * _L, _L)]
          accn = accn + hn_v[pl.ds(l * 256 + j * _L, _L)]
        red_v[pl.ds(j * _L, _L)] = accp
        red_v[pl.ds(256 + j * _L, _L)] = accn
        return 0
      lax.fori_loop(0, 16, lr, 0)

      pltpu.sync_copy(red_v, shist.at[r, pl.ds(sid * 512, 512)])
      plsc.subcore_barrier()
      pltpu.sync_copy(shist.at[r], allh_v)

      def gm(j, _):
        accp = zi
        accn = zi
        for ss in range(_NSUB):
          accp = accp + allh_v[pl.ds(ss * 512 + j * _L, _L)]
          accn = accn + allh_v[pl.ds(ss * 512 + 256 + j * _L, _L)]
        gh_v[pl.ds(j * _L, _L)] = accp
        gh_v[pl.ds(256 + j * _L, _L)] = accn
        return 0
      lax.fori_loop(0, 16, gm, 0)

      noff = 0 if r == 0 else 256

      def htot(off):
        def tb(j, acc):
          return acc + gh_v[pl.ds(off + j * _L, _L)]
        return jnp.sum(lax.fori_loop(0, 16, tb, zi))

      total_p = htot(0)
      total_n = total_p if r == 0 else htot(256)

      def mb_desc(j, carry):
        cnt, hsum, rowpref = carry
        h = gh_v[pl.ds(j * _L, _L)]
        cums = jnp.cumsum(h)
        pref_lt = rowpref + cums - h
        m = (total_p - pref_lt) >= kp
        return (cnt + plsc.all_reduce_population_count(m),
                hsum + jnp.sum(jnp.where(m, h, 0)),
                rowpref + jnp.sum(h))
      cntp, hsump, _ = lax.fori_loop(
          0, 16, mb_desc, (zi, jnp.int32(0), jnp.int32(0)))
      bp = cntp - 1
      abovep = total_p - hsump

      def mb_asc(j, carry):
        cnt, hsum, rowpref = carry
        h = gh_v[pl.ds(noff + j * _L, _L)]
        cums = jnp.cumsum(h)
        m = (rowpref + cums) >= kn
        return (cnt + plsc.all_reduce_population_count(m),
                hsum + jnp.sum(jnp.where(m, h, 0)),
                rowpref + jnp.sum(h))
      cntn, hsumn, _ = lax.fori_loop(
          0, 16, mb_asc, (zi, jnp.int32(0), jnp.int32(0)))
      bn = 256 - cntn
      belown = total_n - hsumn

      kp = kp - abovep
      pp = pp | (bp << sh)
      kn = kn - belown
      pn = pn | (bn << sh)

    def cnt_body(i, carry):
      cp, cn, sg, st = carry
      v = ts_v[pl.ds(i * _L, _L)]
      bits = lax.bitcast_convert_type(v, jnp.int32)
      cp = cp + (bits == pp).astype(jnp.int32)
      cn = cn + (bits == pn).astype(jnp.int32)
      sg = sg + jnp.where(bits > pp, v, 0.0)
      st = st + v
      return (cp, cn, sg, st)
    cpv, cnv, sgv, stv = lax.fori_loop(0, nv, cnt_body, (zi, zi, zf, zf))
    cposf = jnp.sum(cpv).astype(jnp.float32)
    cnegf = jnp.sum(cnv).astype(jnp.float32)
    sgt = jnp.sum(sgv)
    stot = jnp.sum(stv)

    stats = (cposf * (lane == 0).astype(jnp.float32)
             + cnegf * (lane == 1).astype(jnp.float32)
             + sgt * (lane == 2).astype(jnp.float32)
             + stot * (lane == 3).astype(jnp.float32))
    st_v[...] = stats
    pltpu.sync_copy(st_v, sstat.at[pl.ds(sid * _L, _L)])
    plsc.subcore_barrier()
    pltpu.sync_copy(sstat, alls_v)

    colp = plsc.load_gather(alls_v, [lane * _L])
    coln = plsc.load_gather(alls_v, [lane * _L + 1])
    colg = plsc.load_gather(alls_v, [lane * _L + 2])
    cols = plsc.load_gather(alls_v, [lane * _L + 3])
    beforem = (lane < sid).astype(jnp.float32)
    eqpre_p = jnp.sum(colp * beforem)
    eqpre_n = jnp.sum(coln * beforem)
    qpos = (kp.astype(jnp.float32) - eqpre_p).astype(jnp.int32)
    qneg = (kn.astype(jnp.float32) - eqpre_n).astype(jnp.int32)

    tot_g = jnp.sum(colg)
    tot_s = jnp.sum(cols)
    tval = lax.bitcast_convert_type(pp, jnp.float32)
    fgv = tot_g + kp.astype(jnp.float32) * tval
    sdv = tot_s * jnp.float32(1.0 / n)
    outv = (fgv * (lane == 0).astype(jnp.float32)
            + sdv * (lane == 1).astype(jnp.float32))
    scal_v[...] = outv

    @pl.when(sid == 0)
    def _():
      pltpu.sync_copy(scal_v, scal_hbm)

    def mask_body(i, carry):
      lep, len_ = carry
      v = ts_v[pl.ds(i * _L, _L)]
      bits = lax.bitcast_convert_type(v, jnp.int32)
      eqp = bits == pp
      eqn = bits == pn
      cump = jnp.cumsum(eqp.astype(jnp.int32))
      cumn = jnp.cumsum(eqn.astype(jnp.int32))
      pos_sel = (bits > pp) | (eqp & ((lep + cump) <= qpos))
      neg_sel = (bits < pn) | (eqn & ((len_ + cumn) <= qneg))
      pos_v[pl.ds(i * _L, _L)] = (pos_sel & jnp.logical_not(neg_sel)).astype(jnp.float32)
      neg_v[pl.ds(i * _L, _L)] = neg_sel.astype(jnp.float32)
      return (lep + plsc.all_reduce_population_count(eqp),
              len_ + plsc.all_reduce_population_count(eqn))
    lax.fori_loop(0, nv, mask_body, (zi, zi))

    pltpu.sync_copy(pos_v, pos_hbm.at[pl.ds(base, rows)])
    pltpu.sync_copy(neg_v, neg_hbm.at[pl.ds(base, rows)])

  return sel(cls_flat, cent)


def kernel(t_cls_scores, t_centernesses):
  n, nc = t_cls_scores.shape
  k = max(int(n * 0.01), 2)
  joint, posm, negm, scal = _sc_stage(
      t_cls_scores.reshape(-1), t_centernesses.reshape(-1), k, nc)
  return (posm > 0, negm > 0, joint, scal[0], scal[1], joint)


# speedup vs baseline: 3.0526x; 1.0030x over previous
import functools

import jax
import jax.numpy as jnp
from jax import lax
from jax.experimental import pallas as pl
from jax.experimental.pallas import tpu as pltpu
from jax.experimental.pallas import tpu_sc as plsc

_L = 16
_NSUB = 16


def _sc_stage(cls_flat, cent, k, nc):
  n = cls_flat.shape[0] // nc
  rows = n // _NSUB
  nv = rows // _L
  crows = 176
  nchunk = rows // crows
  gpc = crows // _L

  mesh = plsc.VectorSubcoreMesh(
      core_axis_name="c", subcore_axis_name="s", num_cores=1)

  out_type = (
      jax.ShapeDtypeStruct((n,), jnp.float32),
      jax.ShapeDtypeStruct((n,), jnp.float32),
      jax.ShapeDtypeStruct((n,), jnp.float32),
      jax.ShapeDtypeStruct((_L,), jnp.float32),
  )
  scratch = [
      pltpu.VMEM((crows * nc,), jnp.float32),
      pltpu.VMEM((crows * nc,), jnp.float32),
      pltpu.VMEM((crows * nc,), jnp.float32),
      pltpu.VMEM((crows * nc,), jnp.float32),
      pltpu.VMEM((rows,), jnp.float32),
      pltpu.VMEM((rows,), jnp.float32),
      pltpu.VMEM((rows,), jnp.float32),
      pltpu.VMEM((_L * 256,), jnp.int32),
      pltpu.VMEM((_L * 256,), jnp.int32),
      pltpu.VMEM((512,), jnp.int32),
      pltpu.VMEM((_NSUB * 512,), jnp.int32),
      pltpu.VMEM((512,), jnp.int32),
      pltpu.VMEM((_L,), jnp.float32),
      pltpu.VMEM((_NSUB * _L,), jnp.float32),
      pltpu.VMEM((_L,), jnp.float32),
      pltpu.VMEM_SHARED((4, _NSUB * 512), jnp.int32),
      pltpu.VMEM_SHARED((_NSUB * _L,), jnp.float32),
      pltpu.SemaphoreType.DMA,
      pltpu.SemaphoreType.DMA,
      pltpu.SemaphoreType.DMA,
      pltpu.SemaphoreType.DMA,
  ]

  @functools.partial(
      pl.kernel, out_type=out_type, mesh=mesh, scratch_types=scratch,
      compiler_params=pltpu.CompilerParams(needs_layout_passes=False))
  def sel(cls_hbm, cent_hbm, joint_hbm, pos_hbm, neg_hbm, scal_hbm,
          cls_a, cls_b, cls_c, cls_d, cent_v, joint_v, ts_v, hp_v, hn_v,
          red_v, allh_v, gh_v, st_v, alls_v, scal_v, shist, sstat,
          sem_a, sem_b, sem_c, sem_d):
    pos_v = joint_v
    neg_v = cent_v
    sid = lax.axis_index("s")
    base = sid * rows
    lane = lax.iota(jnp.int32, _L)
    ones = jnp.ones((_L,), jnp.int32)
    zi = jnp.zeros((_L,), jnp.int32)
    zf = jnp.zeros((_L,), jnp.float32)

    pltpu.sync_copy(cent_hbm.at[pl.ds(base, rows)], cent_v)
    cidx = [jnp.full((_L,), c, jnp.int32) for c in range(nc)]

    def cls_dma(ch, buf, sem):
      return pltpu.make_async_copy(
          cls_hbm.at[pl.ds((base + ch * crows) * nc, crows * nc)], buf, sem)

    def rowmax_chunk(ch, buf):
      row0 = ch * crows

      def g_body(g, _):
        ridx = (g * _L + lane) * nc
        m = plsc.load_gather(buf, [ridx + cidx[0]])
        for c in range(1, nc):
          m = jnp.maximum(m, plsc.load_gather(buf, [ridx + cidx[c]]))
        ts_v[pl.ds(row0 + g * _L, _L)] = m
        return 0
      lax.fori_loop(0, gpc, g_body, 0)

    bufs = [(cls_a, sem_a), (cls_b, sem_b), (cls_c, sem_c), (cls_d, sem_d)]
    nsuper = (nchunk + 3) // 4
    for ch in range(3):
      cls_dma(ch, *bufs[ch]).start()

    def super_body(i, _):
      for b in range(4):
        ch = i * 4 + b
        buf, sem = bufs[b]

        @pl.when(ch + 3 < nchunk)
        def _():
          nbuf, nsem = bufs[(b + 3) % 4]
          cls_dma(ch + 3, nbuf, nsem).start()

        @pl.when(ch < nchunk)
        def _():
          cls_dma(ch, buf, sem).wait()
          rowmax_chunk(ch, buf)
      return 0
    lax.fori_loop(0, nsuper, super_body, 0)

    def sig_body(i, _):
      o = i * (2 * _L)
      m0 = ts_v[pl.ds(o, _L)]
      m1 = ts_v[pl.ds(o + _L, _L)]
      c0 = cent_v[pl.ds(o, _L)]
      c1 = cent_v[pl.ds(o + _L, _L)]
      s0 = 1.0 / (1.0 + jnp.exp(-m0))
      s1 = 1.0 / (1.0 + jnp.exp(-m1))
      e0 = 1.0 / (1.0 + jnp.exp(-c0))
      e1 = 1.0 / (1.0 + jnp.exp(-c1))
      ts_v[pl.ds(o, _L)] = s0
      ts_v[pl.ds(o + _L, _L)] = s1
      joint_v[pl.ds(o, _L)] = e0 * s0
      joint_v[pl.ds(o + _L, _L)] = e1 * s1
      return 0
    lax.fori_loop(0, nv // 2, sig_body, 0)
    pltpu.sync_copy(joint_v, joint_hbm.at[pl.ds(base, rows)])

    kp = jnp.int32(k)
    kn = jnp.int32(k)
    pp = jnp.int32(0)
    pn = jnp.int32(0)
    for r in range(4):
      sh = 24 - 8 * r

      def zb(i, _):
        hp_v[pl.ds(i * _L, _L)] = zi
        hn_v[pl.ds(i * _L, _L)] = zi
        return 0
      lax.fori_loop(0, 256, zb, 0)

      if r == 0:
        def sc0(i, _):
          bits = lax.bitcast_convert_type(ts_v[pl.ds(i * _L, _L)], jnp.int32)
          byte = (bits >> sh) & 255
          plsc.addupdate_scatter(hp_v, [lane * 256 + byte], ones)
          return 0
        lax.fori_loop(0, nv, sc0, 0)
      else:
        mh = jnp.int32(-(1 << (sh + 8)))
        pph = pp
        pnh = pn

        def scr(i, _):
          bits = lax.bitcast_convert_type(ts_v[pl.ds(i * _L, _L)], jnp.int32)
          byte = (bits >> sh) & 255
          idx = lane * 256 + byte
          hi = bits & mh
          plsc.addupdate_scatter(hp_v, [idx], ones, mask=(hi == pph))
          plsc.addupdate_scatter(hn_v, [idx], ones, mask=(hi == pnh))
          return 0
        lax.fori_loop(0, nv, scr, 0)

      def lr(j, _):
        accp = zi
        accn = zi
        for l in range(_L):
          accp = accp + hp_v[pl.ds(l * 256 + j * _L, _L)]
          accn = accn + hn_v[pl.ds(l * 256 + j * _L, _L)]
        red_v[pl.ds(j * _L, _L)] = accp
        red_v[pl.ds(256 + j * _L, _L)] = accn
        return 0
      lax.fori_loop(0, 16, lr, 0)

      pltpu.sync_copy(red_v, shist.at[r, pl.ds(sid * 512, 512)])
      plsc.subcore_barrier()
      pltpu.sync_copy(shist.at[r], allh_v)

      def gm(j, _):
        accp = zi
        accn = zi
        for ss in range(_NSUB):
          accp = accp + allh_v[pl.ds(ss * 512 + j * _L, _L)]
          accn = accn + allh_v[pl.ds(ss * 512 + 256 + j * _L, _L)]
        gh_v[pl.ds(j * _L, _L)] = accp
        gh_v[pl.ds(256 + j * _L, _L)] = accn
        return 0
      lax.fori_loop(0, 16, gm, 0)

      noff = 0 if r == 0 else 256

      def htot(off):
        def tb(j, acc):
          return acc + gh_v[pl.ds(off + j * _L, _L)]
        return jnp.sum(lax.fori_loop(0, 16, tb, zi))

      total_p = htot(0)
      total_n = total_p if r == 0 else htot(256)

      def mb_desc(j, carry):
        cnt, hsum, rowpref = carry
        h = gh_v[pl.ds(j * _L, _L)]
        cums = jnp.cumsum(h)
        pref_lt = rowpref + cums - h
        m = (total_p - pref_lt) >= kp
        return (cnt + plsc.all_reduce_population_count(m),
                hsum + jnp.sum(jnp.where(m, h, 0)),
                rowpref + jnp.sum(h))
      cntp, hsump, _ = lax.fori_loop(
          0, 16, mb_desc, (zi, jnp.int32(0), jnp.int32(0)))
      bp = cntp - 1
      abovep = total_p - hsump

      def mb_asc(j, carry):
        cnt, hsum, rowpref = carry
        h = gh_v[pl.ds(noff + j * _L, _L)]
        cums = jnp.cumsum(h)
        m = (rowpref + cums) >= kn
        return (cnt + plsc.all_reduce_population_count(m),
                hsum + jnp.sum(jnp.where(m, h, 0)),
                rowpref + jnp.sum(h))
      cntn, hsumn, _ = lax.fori_loop(
          0, 16, mb_asc, (zi, jnp.int32(0), jnp.int32(0)))
      bn = 256 - cntn
      belown = total_n - hsumn

      kp = kp - abovep
      pp = pp | (bp << sh)
      kn = kn - belown
      pn = pn | (bn << sh)

    def cnt_body(i, carry):
      cp, cn, sg, st = carry
      v = ts_v[pl.ds(i * _L, _L)]
      bits = lax.bitcast_convert_type(v, jnp.int32)
      cp = cp + (bits == pp).astype(jnp.int32)
      cn = cn + (bits == pn).astype(jnp.int32)
      sg = sg + jnp.where(bits > pp, v, 0.0)
      st = st + v
      return (cp, cn, sg, st)
    cpv, cnv, sgv, stv = lax.fori_loop(0, nv, cnt_body, (zi, zi, zf, zf))
    cposf = jnp.sum(cpv).astype(jnp.float32)
    cnegf = jnp.sum(cnv).astype(jnp.float32)
    sgt = jnp.sum(sgv)
    stot = jnp.sum(stv)

    stats = (cposf * (lane == 0).astype(jnp.float32)
             + cnegf * (lane == 1).astype(jnp.float32)
             + sgt * (lane == 2).astype(jnp.float32)
             + stot * (lane == 3).astype(jnp.float32))
    st_v[...] = stats
    pltpu.sync_copy(st_v, sstat.at[pl.ds(sid * _L, _L)])
    plsc.subcore_barrier()
    pltpu.sync_copy(sstat, alls_v)

    colp = plsc.load_gather(alls_v, [lane * _L])
    coln = plsc.load_gather(alls_v, [lane * _L + 1])
    colg = plsc.load_gather(alls_v, [lane * _L + 2])
    cols = plsc.load_gather(alls_v, [lane * _L + 3])
    beforem = (lane < sid).astype(jnp.float32)
    eqpre_p = jnp.sum(colp * beforem)
    eqpre_n = jnp.sum(coln * beforem)
    qpos = (kp.astype(jnp.float32) - eqpre_p).astype(jnp.int32)
    qneg = (kn.astype(jnp.float32) - eqpre_n).astype(jnp.int32)

    tot_g = jnp.sum(colg)
    tot_s = jnp.sum(cols)
    tval = lax.bitcast_convert_type(pp, jnp.float32)
    fgv = tot_g + kp.astype(jnp.float32) * tval
    sdv = tot_s * jnp.float32(1.0 / n)
    outv = (fgv * (lane == 0).astype(jnp.float32)
            + sdv * (lane == 1).astype(jnp.float32))
    scal_v[...] = outv

    @pl.when(sid == 0)
    def _():
      pltpu.sync_copy(scal_v, scal_hbm)

    def mask_body(i, carry):
      lep, len_ = carry
      v = ts_v[pl.ds(i * _L, _L)]
      bits = lax.bitcast_convert_type(v, jnp.int32)
      eqp = bits == pp
      eqn = bits == pn
      cump = jnp.cumsum(eqp.astype(jnp.int32))
      cumn = jnp.cumsum(eqn.astype(jnp.int32))
      pos_sel = (bits > pp) | (eqp & ((lep + cump) <= qpos))
      neg_sel = (bits < pn) | (eqn & ((len_ + cumn) <= qneg))
      pos_v[pl.ds(i * _L, _L)] = (pos_sel & jnp.logical_not(neg_sel)).astype(jnp.float32)
      neg_v[pl.ds(i * _L, _L)] = neg_sel.astype(jnp.float32)
      return (lep + plsc.all_reduce_population_count(eqp),
              len_ + plsc.all_reduce_population_count(eqn))
    lax.fori_loop(0, nv, mask_body, (zi, zi))

    pltpu.sync_copy(pos_v, pos_hbm.at[pl.ds(base, rows)])
    pltpu.sync_copy(neg_v, neg_hbm.at[pl.ds(base, rows)])

  return sel(cls_flat, cent)


def kernel(t_cls_scores, t_centernesses):
  n, nc = t_cls_scores.shape
  k = max(int(n * 0.01), 2)
  joint, posm, negm, scal = _sc_stage(
      t_cls_scores.reshape(-1), t_centernesses.reshape(-1), k, nc)
  return (posm > 0, negm > 0, joint, scal[0], scal[1], joint)
